# baseline (speedup 1.0000x reference)
import functools

import jax
import jax.numpy as jnp
from jax import lax
from jax.experimental import pallas as pl
from jax.experimental.pallas import tpu as pltpu
from jax.experimental.pallas import tpu_sc as plsc

_N_F = 7
_N_P = 16
_COLS = _N_F + _N_P
_D = 64
_CHUNK = 512
_LANES = 16
_GROUPS = _CHUNK // _LANES
_N_TILES = 32
_IDX_SLICE = 128


def _table_body(f_ref, p_ref, wf_ref, wp_ref, b_ref, t_ref):
    wf = jnp.dot(f_ref[...], wf_ref[...], preferred_element_type=jnp.float32)
    wp = jnp.dot(p_ref[...], wp_ref[...], preferred_element_type=jnp.float32)
    t_ref[...] = wf[:, None, :] + wp[None, :, :] + b_ref[...]


def _build_table(f_table, p_table, fc_w, fc_b):
    t3 = pl.pallas_call(
        _table_body,
        out_shape=jax.ShapeDtypeStruct((_N_F, _N_P, _D), jnp.float32),
    )(
        f_table,
        p_table,
        fc_w[:, :_N_P].T,
        fc_w[:, _N_P:].T,
        fc_b.reshape(1, 1, _D),
    )
    return t3.reshape(_N_F * _N_P, _D)


def _argtree(vals, idxs):
    while len(vals) > 1:
        nv, ni = [], []
        for k in range(0, len(vals) - 1, 2):
            ge = vals[k] >= vals[k + 1]
            nv.append(jnp.maximum(vals[k], vals[k + 1]))
            ni.append(jnp.where(ge, idxs[k], idxs[k + 1]))
        if len(vals) % 2:
            nv.append(vals[-1])
            ni.append(idxs[-1])
        vals, idxs = nv, ni
    return idxs[0]


def _edge_embed(edge, t, n):
    assert n % _CHUNK == 0
    n_chunks = n // _CHUNK
    mesh = plsc.VectorSubcoreMesh(core_axis_name="c", subcore_axis_name="s")

    @functools.partial(
        pl.kernel,
        mesh=mesh,
        compiler_params=pltpu.CompilerParams(
            needs_layout_passes=False, use_tc_tiling_on_sc=False),
        out_type=jax.ShapeDtypeStruct((n, _D), jnp.float32),
        scratch_types=[
            pltpu.VMEM((_CHUNK, _COLS), jnp.float32),
            pltpu.VMEM((_CHUNK, _COLS), jnp.float32),
            pltpu.VMEM((_CHUNK, _D), jnp.float32),
            pltpu.VMEM((_CHUNK, _D), jnp.float32),
            pltpu.VMEM((_CHUNK,), jnp.int32),
            pltpu.VMEM((_CHUNK,), jnp.int32),
            pltpu.SemaphoreType.DMA,
            pltpu.SemaphoreType.DMA,
            pltpu.SemaphoreType.DMA,
            pltpu.SemaphoreType.DMA,
            pltpu.SemaphoreType.DMA,
            pltpu.SemaphoreType.DMA,
        ],
    )
    def run(edge_hbm, t_hbm, out_hbm, ev0, ev1, ov0, ov1, ci0, ci1,
            si0, si1, sg0, sg1, so0, so1):
        cid = lax.axis_index("c")
        sid = lax.axis_index("s")
        wid = sid * 2 + cid
        my_n = (n_chunks - wid + _N_TILES - 1) // _N_TILES
        edge_bufs = (ev0, ev1)
        out_bufs = (ov0, ov1)
        idx_bufs = (ci0, ci1)
        sin = (si0, si1)
        sgat = (sg0, sg1)
        sout = (so0, so1)

        def in_slice(i):
            return edge_hbm.at[pl.ds((wid + i * _N_TILES) * _CHUNK, _CHUNK)]

        def out_slice(i):
            return out_hbm.at[pl.ds((wid + i * _N_TILES) * _CHUNK, _CHUNK)]

        def compute_idx(edge_v, cidx_v):
            def group_body(g, c2):
                rows = g * _LANES + lax.iota(jnp.int32, _LANES)

                def col(c):
                    return plsc.load_gather(
                        edge_v, [rows, jnp.full((_LANES,), c, jnp.int32)])

                fidx = _argtree(
                    [col(c) for c in range(_N_F)],
                    [jnp.full((_LANES,), c, jnp.int32) for c in range(_N_F)])
                pidx = _argtree(
                    [col(_N_F + c) for c in range(_N_P)],
                    [jnp.full((_LANES,), c, jnp.int32) for c in range(_N_P)])
                cidx_v[pl.ds(g * _LANES, _LANES)] = fidx * _N_P + pidx
                return c2

            lax.fori_loop(0, _GROUPS, group_body, 0, unroll=4)

        def gather_rows(cidx_v, out_v, sem):
            for j in range(0, _CHUNK, _IDX_SLICE):
                pltpu.async_copy(
                    t_hbm.at[cidx_v.at[pl.ds(j, _IDX_SLICE)]],
                    out_v.at[pl.ds(j, _IDX_SLICE)], sem)
            for j in range(0, _CHUNK, _IDX_SLICE):
                pltpu.make_async_copy(
                    t_hbm.at[cidx_v.at[pl.ds(j, _IDX_SLICE)]],
                    out_v.at[pl.ds(j, _IDX_SLICE)], sem).wait()

        for b in range(2):
            @pl.when(b < my_n)
            def _():
                pltpu.async_copy(in_slice(b), edge_bufs[b], sin[b])

        def outer(j, carry):
            for b in range(2):
                i = 2 * j + b

                @pl.when(i < my_n)
                def _():
                    pltpu.make_async_copy(in_slice(i), edge_bufs[b], sin[b]).wait()
                    compute_idx(edge_bufs[b], idx_bufs[b])

                    @pl.when(i >= 2)
                    def _():
                        pltpu.make_async_copy(
                            out_bufs[b], out_slice(i - 2), sout[b]).wait()

                    gather_rows(idx_bufs[b], out_bufs[b], sgat[b])
                    pltpu.async_copy(out_bufs[b], out_slice(i), sout[b])

                    @pl.when(i + 2 < my_n)
                    def _():
                        pltpu.async_copy(in_slice(i + 2), edge_bufs[b], sin[b])
            return carry

        lax.fori_loop(0, (my_n + 1) // 2, outer, 0)

        for b in range(2):
            @pl.when(b < my_n)
            def _():
                pltpu.make_async_copy(out_bufs[b], out_slice(b), sout[b]).wait()

    return run(edge, t)


def kernel(edge, p_table, f_table, fc_w, fc_b):
    n = edge.shape[0]
    t = _build_table(f_table, p_table, fc_w, fc_b)
    return _edge_embed(edge, t, n)


# speedup vs baseline: 1.3603x; 1.3603x over previous
import functools

import jax
import jax.numpy as jnp
from jax import lax
from jax.experimental import pallas as pl
from jax.experimental.pallas import tpu as pltpu
from jax.experimental.pallas import tpu_sc as plsc

_N_F = 7
_N_P = 16
_COLS = _N_F + _N_P
_D = 64
_CHUNK = 512
_LANES = 16
_GROUPS = _CHUNK // _LANES
_N_TILES = 32
_IDX_SLICE = 128


def _table_body(f_ref, p_ref, wf_ref, wp_ref, b_ref, t_ref):
    wf = jnp.dot(f_ref[...], wf_ref[...], preferred_element_type=jnp.float32)
    wp = jnp.dot(p_ref[...], wp_ref[...], preferred_element_type=jnp.float32)
    t_ref[...] = wf[:, None, :] + wp[None, :, :] + b_ref[...]


def _build_table(f_table, p_table, fc_w, fc_b):
    t3 = pl.pallas_call(
        _table_body,
        out_shape=jax.ShapeDtypeStruct((_N_F, _N_P, _D), jnp.float32),
    )(
        f_table,
        p_table,
        fc_w[:, :_N_P].T,
        fc_w[:, _N_P:].T,
        fc_b.reshape(1, 1, _D),
    )
    return t3.reshape(_N_F * _N_P, _D)


def _argtree(vals, idxs):
    while len(vals) > 1:
        nv, ni = [], []
        for k in range(0, len(vals) - 1, 2):
            ge = vals[k] >= vals[k + 1]
            nv.append(jnp.maximum(vals[k], vals[k + 1]))
            ni.append(jnp.where(ge, idxs[k], idxs[k + 1]))
        if len(vals) % 2:
            nv.append(vals[-1])
            ni.append(idxs[-1])
        vals, idxs = nv, ni
    return idxs[0]


def _edge_embed(edge, t, n):
    assert n % _CHUNK == 0
    n_chunks = n // _CHUNK
    mesh = plsc.VectorSubcoreMesh(core_axis_name="c", subcore_axis_name="s")

    @functools.partial(
        pl.kernel,
        mesh=mesh,
        compiler_params=pltpu.CompilerParams(
            needs_layout_passes=False, use_tc_tiling_on_sc=False),
        out_type=jax.ShapeDtypeStruct((n, _D), jnp.float32),
        scratch_types=[
            pltpu.VMEM((_CHUNK, _COLS), jnp.float32),
            pltpu.VMEM((_CHUNK, _COLS), jnp.float32),
            pltpu.VMEM((_CHUNK, _D), jnp.float32),
            pltpu.VMEM((_CHUNK, _D), jnp.float32),
            pltpu.VMEM((_CHUNK,), jnp.int32),
            pltpu.VMEM((_CHUNK,), jnp.int32),
            pltpu.VMEM_SHARED((_N_F * _N_P, _D), jnp.float32),
            pltpu.SemaphoreType.DMA,
            pltpu.SemaphoreType.DMA,
            pltpu.SemaphoreType.DMA,
            pltpu.SemaphoreType.DMA,
            pltpu.SemaphoreType.DMA,
            pltpu.SemaphoreType.DMA,
        ],
    )
    def run(edge_hbm, t_hbm, out_hbm, ev0, ev1, ov0, ov1, ci0, ci1, t_sh,
            si0, si1, sg0, sg1, so0, so1):
        cid = lax.axis_index("c")
        sid = lax.axis_index("s")
        wid = sid * 2 + cid

        @pl.when(sid == 0)
        def _():
            pltpu.sync_copy(t_hbm, t_sh)

        plsc.subcore_barrier()
        my_n = (n_chunks - wid + _N_TILES - 1) // _N_TILES
        edge_bufs = (ev0, ev1)
        out_bufs = (ov0, ov1)
        idx_bufs = (ci0, ci1)
        sin = (si0, si1)
        sgat = (sg0, sg1)
        sout = (so0, so1)

        def in_slice(i):
            return edge_hbm.at[pl.ds((wid + i * _N_TILES) * _CHUNK, _CHUNK)]

        def out_slice(i):
            return out_hbm.at[pl.ds((wid + i * _N_TILES) * _CHUNK, _CHUNK)]

        def compute_idx(edge_v, cidx_v):
            def group_body(g, c2):
                rows = g * _LANES + lax.iota(jnp.int32, _LANES)

                def col(c):
                    return plsc.load_gather(
                        edge_v, [rows, jnp.full((_LANES,), c, jnp.int32)])

                fidx = _argtree(
                    [col(c) for c in range(_N_F)],
                    [jnp.full((_LANES,), c, jnp.int32) for c in range(_N_F)])
                pidx = _argtree(
                    [col(_N_F + c) for c in range(_N_P)],
                    [jnp.full((_LANES,), c, jnp.int32) for c in range(_N_P)])
                cidx_v[pl.ds(g * _LANES, _LANES)] = fidx * _N_P + pidx
                return c2

            lax.fori_loop(0, _GROUPS, group_body, 0, unroll=4)

        def gather_rows(cidx_v, out_v, sem):
            for j in range(0, _CHUNK, _IDX_SLICE):
                pltpu.async_copy(
                    t_sh.at[cidx_v.at[pl.ds(j, _IDX_SLICE)]],
                    out_v.at[pl.ds(j, _IDX_SLICE)], sem)
            for j in range(0, _CHUNK, _IDX_SLICE):
                pltpu.make_async_copy(
                    t_sh.at[cidx_v.at[pl.ds(j, _IDX_SLICE)]],
                    out_v.at[pl.ds(j, _IDX_SLICE)], sem).wait()

        for b in range(2):
            @pl.when(b < my_n)
            def _():
                pltpu.async_copy(in_slice(b), edge_bufs[b], sin[b])

        def outer(j, carry):
            for b in range(2):
                i = 2 * j + b

                @pl.when(i < my_n)
                def _():
                    pltpu.make_async_copy(in_slice(i), edge_bufs[b], sin[b]).wait()
                    compute_idx(edge_bufs[b], idx_bufs[b])

                    @pl.when(i >= 2)
                    def _():
                        pltpu.make_async_copy(
                            out_bufs[b], out_slice(i - 2), sout[b]).wait()

                    gather_rows(idx_bufs[b], out_bufs[b], sgat[b])
                    pltpu.async_copy(out_bufs[b], out_slice(i), sout[b])

                    @pl.when(i + 2 < my_n)
                    def _():
                        pltpu.async_copy(in_slice(i + 2), edge_bufs[b], sin[b])
            return carry

        lax.fori_loop(0, (my_n + 1) // 2, outer, 0)

        for b in range(2):
            @pl.when(b < my_n)
            def _():
                pltpu.make_async_copy(out_bufs[b], out_slice(b), sout[b]).wait()

    return run(edge, t)


def kernel(edge, p_table, f_table, fc_w, fc_b):
    n = edge.shape[0]
    t = _build_table(f_table, p_table, fc_w, fc_b)
    return _edge_embed(edge, t, n)
